# baseline (speedup 1.0000x reference)
import functools

import jax
import jax.numpy as jnp
from jax import lax
from jax.experimental import pallas as pl
from jax.experimental.pallas import tpu as pltpu
from jax.experimental.pallas import tpu_sc as plsc

EMB = 88
ATOM_VOCABS = (119, 4, 12, 12, 10, 6, 6, 2, 2)
BOND_VOCABS = (5, 6, 2)

BN = 10000
BE = 16000


def _node_body(x_ref, t_ref, o_ref):
    xf = x_ref[...].astype(jnp.float32)
    acc = jnp.zeros((BN, EMB), dtype=jnp.float32)
    for i, v in enumerate(ATOM_VOCABS):
        sel = (jax.lax.broadcasted_iota(jnp.int32, (9, 1), 0) == i
               ).astype(jnp.float32)
        idx = jnp.dot(xf, sel, preferred_element_type=jnp.float32
                      ).astype(jnp.int32)
        cols = jax.lax.broadcasted_iota(jnp.int32, (BN, v), 1)
        oh = (cols == idx).astype(jnp.float32)
        acc = acc + jnp.dot(oh, t_ref[i, :v, :],
                            preferred_element_type=jnp.float32)
    o_ref[...] = acc


def _edge_body(e_ref, t_ref, o_ref):
    ef = e_ref[...].astype(jnp.float32)
    r = jax.lax.broadcasted_iota(jnp.int32, (3, 1), 0)
    w = jnp.where(r == 0, 12.0, jnp.where(r == 1, 2.0, 1.0)).astype(jnp.float32)
    c = jnp.dot(ef, w, preferred_element_type=jnp.float32
                ).astype(jnp.int32)
    rows0 = jax.lax.broadcasted_iota(jnp.int32, (64, 5), 0)
    k0 = jax.lax.broadcasted_iota(jnp.int32, (64, 5), 1)
    a0 = ((rows0 // 12) == k0).astype(jnp.float32)
    rows1 = jax.lax.broadcasted_iota(jnp.int32, (64, 6), 0)
    k1 = jax.lax.broadcasted_iota(jnp.int32, (64, 6), 1)
    a1 = (((rows1 // 2) % 6) == k1).astype(jnp.float32)
    rows2 = jax.lax.broadcasted_iota(jnp.int32, (64, 2), 0)
    k2 = jax.lax.broadcasted_iota(jnp.int32, (64, 2), 1)
    a2 = ((rows2 % 2) == k2).astype(jnp.float32)
    combo = (jnp.dot(a0, t_ref[0, :5, :], preferred_element_type=jnp.float32)
             + jnp.dot(a1, t_ref[1, :6, :], preferred_element_type=jnp.float32)
             + jnp.dot(a2, t_ref[2, :2, :], preferred_element_type=jnp.float32))
    cols = jax.lax.broadcasted_iota(jnp.int32, (BE, 64), 1)
    oh = (cols == c).astype(jnp.float32)
    o_ref[...] = jnp.dot(oh, combo, preferred_element_type=jnp.float32)


def _combo_body(t_ref, o_ref):
    rows0 = jax.lax.broadcasted_iota(jnp.int32, (64, 5), 0)
    k0 = jax.lax.broadcasted_iota(jnp.int32, (64, 5), 1)
    a0 = ((rows0 // 12) == k0).astype(jnp.float32)
    rows1 = jax.lax.broadcasted_iota(jnp.int32, (64, 6), 0)
    k1 = jax.lax.broadcasted_iota(jnp.int32, (64, 6), 1)
    a1 = (((rows1 // 2) % 6) == k1).astype(jnp.float32)
    rows2 = jax.lax.broadcasted_iota(jnp.int32, (64, 2), 0)
    k2 = jax.lax.broadcasted_iota(jnp.int32, (64, 2), 1)
    a2 = ((rows2 % 2) == k2).astype(jnp.float32)
    o_ref[...] = (
        jnp.dot(a0, t_ref[0, :5, :], preferred_element_type=jnp.float32)
        + jnp.dot(a1, t_ref[1, :6, :], preferred_element_type=jnp.float32)
        + jnp.dot(a2, t_ref[2, :2, :], preferred_element_type=jnp.float32))


N_EDGES_SC = 800000
NW = 32
EPW = N_EDGES_SC // NW
CE = 512
EMB_PAD = 128
NGR = CE // 128
NFOLD = CE // 16
NCH = -(-EPW // CE)


def _sc_edge_body(e_hbm, combo_hbm, o_hbm, ebuf, cbuf, obuf, sem):
    wid = lax.axis_index("s") * 2 + lax.axis_index("c")
    start = wid * EPW

    def chunk(k, carry):
        b = jnp.minimum(start + k * CE, start + (EPW - CE))
        for j in range(3):
            pltpu.sync_copy(e_hbm.at[pl.ds(j * N_EDGES_SC + b, CE)],
                            ebuf.at[pl.ds(j * CE, CE)])

        def fold(i, c2):
            v0 = ebuf[pl.ds(i * 16, 16)]
            v1 = ebuf[pl.ds(CE + i * 16, 16)]
            v2 = ebuf[pl.ds(2 * CE + i * 16, 16)]
            cbuf[pl.ds(i * 16, 16)] = v0 * 12 + v1 * 2 + v2
            return c2

        lax.fori_loop(0, NFOLD, fold, 0)
        cps = [
            pltpu.async_copy(
                combo_hbm.at[cbuf.at[pl.ds(j * 128, 128)]],
                obuf.at[pl.ds(j * 128, 128)], sem)
            for j in range(NGR)
        ]
        for cp in cps:
            cp.wait()
        pltpu.sync_copy(obuf, o_hbm.at[pl.ds(b, CE)])
        return carry

    lax.fori_loop(0, NCH, chunk, 0)


def _sc_edge(e_flat, combo):
    mesh = plsc.VectorSubcoreMesh(core_axis_name="c", subcore_axis_name="s")
    f = pl.kernel(
        _sc_edge_body,
        out_type=jax.ShapeDtypeStruct((N_EDGES_SC, EMB_PAD), jnp.float32),
        mesh=mesh,
        scratch_types=[
            pltpu.VMEM((CE * 3,), jnp.int32),
            pltpu.VMEM((CE,), jnp.int32),
            pltpu.VMEM((CE, EMB_PAD), jnp.float32),
            pltpu.SemaphoreType.DMA,
        ],
    )
    return f(e_flat, combo)


@jax.jit
def kernel(x, edge_attr, atom_tables, bond_tables):
    n = x.shape[0]
    e = edge_attr.shape[0]
    node_emb = pl.pallas_call(
        _node_body,
        grid=(n // BN,),
        in_specs=[
            pl.BlockSpec((BN, x.shape[1]), lambda i: (i, 0)),
            pl.BlockSpec(atom_tables.shape, lambda i: (0, 0, 0)),
        ],
        out_specs=pl.BlockSpec((BN, EMB), lambda i: (i, 0)),
        out_shape=jax.ShapeDtypeStruct((n, EMB), jnp.float32),
    )(x, atom_tables)
    bt_pad = jnp.pad(bond_tables, ((0, 0), (0, 0), (0, EMB_PAD - EMB)))
    combo = pl.pallas_call(
        _combo_body,
        in_specs=[pl.BlockSpec(bt_pad.shape, lambda: (0, 0, 0))],
        out_specs=pl.BlockSpec((64, EMB_PAD), lambda: (0, 0)),
        out_shape=jax.ShapeDtypeStruct((64, EMB_PAD), jnp.float32),
    )(bt_pad)
    edge_emb = _sc_edge(edge_attr.T.reshape(-1), combo)[:, :EMB]
    return (node_emb, edge_emb)


# speedup vs baseline: 1.7764x; 1.7764x over previous
import jax
import jax.numpy as jnp
from jax.experimental import pallas as pl

EMB = 88
ATOM_VOCABS = (119, 4, 12, 12, 10, 6, 6, 2, 2)
BOND_VOCABS = (5, 6, 2)

N_NODES_K = 50000
BN = 5000
BE = 16000


def _node_body(x_ref, t_ref, o_ref):
    xf = x_ref[...].astype(jnp.float32)
    acc = jnp.zeros((BN, EMB), dtype=jnp.float32)
    for i, v in enumerate(ATOM_VOCABS):
        sel = (jax.lax.broadcasted_iota(jnp.int32, (9, 1), 0) == i
               ).astype(jnp.float32)
        idx = jnp.dot(xf, sel, preferred_element_type=jnp.float32
                      ).astype(jnp.int32)
        cols = jax.lax.broadcasted_iota(jnp.int32, (BN, v), 1)
        oh = (cols == idx).astype(jnp.float32)
        acc = acc + jnp.dot(oh, t_ref[i, :v, :],
                            preferred_element_type=jnp.float32)
    o_ref[...] = acc


def _edge_body(e_ref, t_ref, o_ref):
    ef = e_ref[...].astype(jnp.float32)
    r = jax.lax.broadcasted_iota(jnp.int32, (3, 1), 0)
    w = jnp.where(r == 0, 12.0, jnp.where(r == 1, 2.0, 1.0)).astype(jnp.float32)
    c = jnp.dot(ef, w, preferred_element_type=jnp.float32
                ).astype(jnp.int32)
    rows0 = jax.lax.broadcasted_iota(jnp.int32, (64, 5), 0)
    k0 = jax.lax.broadcasted_iota(jnp.int32, (64, 5), 1)
    a0 = ((rows0 // 12) == k0).astype(jnp.float32)
    rows1 = jax.lax.broadcasted_iota(jnp.int32, (64, 6), 0)
    k1 = jax.lax.broadcasted_iota(jnp.int32, (64, 6), 1)
    a1 = (((rows1 // 2) % 6) == k1).astype(jnp.float32)
    rows2 = jax.lax.broadcasted_iota(jnp.int32, (64, 2), 0)
    k2 = jax.lax.broadcasted_iota(jnp.int32, (64, 2), 1)
    a2 = ((rows2 % 2) == k2).astype(jnp.float32)
    combo = (jnp.dot(a0, t_ref[0, :5, :], preferred_element_type=jnp.float32)
             + jnp.dot(a1, t_ref[1, :6, :], preferred_element_type=jnp.float32)
             + jnp.dot(a2, t_ref[2, :2, :], preferred_element_type=jnp.float32))
    cols = jax.lax.broadcasted_iota(jnp.int32, (BE, 64), 1)
    oh = (cols == c).astype(jnp.float32)
    o_ref[...] = jnp.dot(oh, combo, preferred_element_type=jnp.float32)


def _fused_body(x_ref, e_ref, at_ref, bt_ref, on_ref, oe_ref):
    i = pl.program_id(0)

    @pl.when(i < N_NODES_K // BN)
    def _():
        _node_body(x_ref, at_ref, on_ref)

    _edge_body(e_ref, bt_ref, oe_ref)


@jax.jit
def kernel(x, edge_attr, atom_tables, bond_tables):
    n = x.shape[0]
    e = edge_attr.shape[0]
    nb = n // BN

    node_emb, edge_emb = pl.pallas_call(
        _fused_body,
        grid=(e // BE,),
        in_specs=[
            pl.BlockSpec((BN, x.shape[1]), lambda i: (jnp.minimum(i, nb - 1), 0)),
            pl.BlockSpec((BE, edge_attr.shape[1]), lambda i: (i, 0)),
            pl.BlockSpec(atom_tables.shape, lambda i: (0, 0, 0)),
            pl.BlockSpec(bond_tables.shape, lambda i: (0, 0, 0)),
        ],
        out_specs=[
            pl.BlockSpec((BN, EMB), lambda i: (jnp.minimum(i, nb - 1), 0)),
            pl.BlockSpec((BE, EMB), lambda i: (i, 0)),
        ],
        out_shape=[
            jax.ShapeDtypeStruct((n, EMB), jnp.float32),
            jax.ShapeDtypeStruct((e, EMB), jnp.float32),
        ],
    )(x, edge_attr, atom_tables, bond_tables)
    return (node_emb, edge_emb)


# speedup vs baseline: 1.8642x; 1.0494x over previous
import jax
import jax.numpy as jnp
from jax.experimental import pallas as pl

EMB = 88
ATOM_VOCABS = (119, 4, 12, 12, 10, 6, 6, 2, 2)
BOND_VOCABS = (5, 6, 2)

BN = 10000
BE = 16000


def _node_body(x_ref, t_ref, o_ref):
    xf = x_ref[...].astype(jnp.float32)
    acc = jnp.zeros((BN, EMB), dtype=jnp.float32)
    for i, v in enumerate(ATOM_VOCABS):
        sel = (jax.lax.broadcasted_iota(jnp.int32, (9, 1), 0) == i
               ).astype(jnp.float32)
        idx = jnp.dot(xf, sel, preferred_element_type=jnp.float32
                      ).astype(jnp.int32)
        cols = jax.lax.broadcasted_iota(jnp.int32, (BN, v), 1)
        oh = (cols == idx).astype(jnp.float32)
        acc = acc + jnp.dot(oh, t_ref[i, :v, :],
                            preferred_element_type=jnp.float32)
    o_ref[...] = acc


def _edge_body(e_ref, t_ref, o_ref):
    ef = e_ref[...].astype(jnp.float32)
    r = jax.lax.broadcasted_iota(jnp.int32, (3, 1), 0)
    w = jnp.where(r == 0, 12.0, jnp.where(r == 1, 2.0, 1.0)).astype(jnp.float32)
    c = jnp.dot(ef, w, preferred_element_type=jnp.float32
                ).astype(jnp.int32)
    rows0 = jax.lax.broadcasted_iota(jnp.int32, (64, 5), 0)
    k0 = jax.lax.broadcasted_iota(jnp.int32, (64, 5), 1)
    a0 = ((rows0 // 12) == k0).astype(jnp.float32)
    rows1 = jax.lax.broadcasted_iota(jnp.int32, (64, 6), 0)
    k1 = jax.lax.broadcasted_iota(jnp.int32, (64, 6), 1)
    a1 = (((rows1 // 2) % 6) == k1).astype(jnp.float32)
    rows2 = jax.lax.broadcasted_iota(jnp.int32, (64, 2), 0)
    k2 = jax.lax.broadcasted_iota(jnp.int32, (64, 2), 1)
    a2 = ((rows2 % 2) == k2).astype(jnp.float32)
    combo = (jnp.dot(a0, t_ref[0, :5, :], preferred_element_type=jnp.float32)
             + jnp.dot(a1, t_ref[1, :6, :], preferred_element_type=jnp.float32)
             + jnp.dot(a2, t_ref[2, :2, :], preferred_element_type=jnp.float32))
    cols = jax.lax.broadcasted_iota(jnp.int32, (BE, 64), 1)
    oh = (cols == c).astype(jnp.float32)
    o_ref[...] = jnp.dot(oh, combo, preferred_element_type=jnp.float32)


@jax.jit
def kernel(x, edge_attr, atom_tables, bond_tables):
    n = x.shape[0]
    e = edge_attr.shape[0]
    node_emb = pl.pallas_call(
        _node_body,
        grid=(n // BN,),
        in_specs=[
            pl.BlockSpec((BN, x.shape[1]), lambda i: (i, 0)),
            pl.BlockSpec(atom_tables.shape, lambda i: (0, 0, 0)),
        ],
        out_specs=pl.BlockSpec((BN, EMB), lambda i: (i, 0)),
        out_shape=jax.ShapeDtypeStruct((n, EMB), jnp.float32),
    )(x, atom_tables)
    edge_emb = pl.pallas_call(
        _edge_body,
        grid=(e // BE,),
        in_specs=[
            pl.BlockSpec((BE, edge_attr.shape[1]), lambda i: (i, 0)),
            pl.BlockSpec(bond_tables.shape, lambda i: (0, 0, 0)),
        ],
        out_specs=pl.BlockSpec((BE, EMB), lambda i: (i, 0)),
        out_shape=jax.ShapeDtypeStruct((e, EMB), jnp.float32),
    )(edge_attr, bond_tables)
    return (node_emb, edge_emb)
